# baseline (speedup 1.0000x reference)
import jax
import jax.numpy as jnp
from jax import lax
from jax.experimental import pallas as pl
from jax.experimental.pallas import tpu as pltpu
from jax.experimental.pallas import tpu_sc as plsc

_B = 320000
_D = 128
_V = 10000
_NC = 2
_NS = 16
_NW = _NC * _NS

_KW = 9984
_C = 128
_PH = 3
_PHE = _KW // _PH
_CPP = _PHE // _C
_PPP = _CPP // 2

_XBASE = _NW * _KW
_XTOT = _B - _XBASE

_VPT = 624
_VREM = _V - _NS * _VPT

_L = 16


def _gather_body(ei_hbm, x_hbm, out_hbm, xs, ib, idx1, r0, r1,
                 g0, g1, s0, s1, isem):
    cid = lax.axis_index("c")
    sid = lax.axis_index("s")
    wid = sid * _NC + cid
    base = wid * _KW

    def slab_start(p):
        src = ei_hbm.at[pl.ds(0, 2), pl.ds(base + p * _PHE, _PHE)]
        pltpu.async_copy(src, ib, isem)

    def slab_wait(p):
        src = ei_hbm.at[pl.ds(0, 2), pl.ds(base + p * _PHE, _PHE)]
        pltpu.make_async_copy(src, ib, isem).wait()

    def extract(p):
        zb = p * _PHE

        def cp(k, carry):
            idx1[pl.ds(zb + k * _L, _L)] = ib[0, pl.ds(k * _L, _L)]
            return carry

        lax.fori_loop(0, _PHE // _L, cp, 0)

    slab_start(0)

    pltpu.sync_copy(x_hbm.at[pl.ds(sid * _VPT, _VPT)],
                    xs.at[pl.ds(sid * _VPT, _VPT)])
    @pl.when(sid == 0)
    def _():
        pltpu.sync_copy(x_hbm.at[pl.ds(_NS * _VPT, _VREM)],
                        xs.at[pl.ds(_NS * _VPT, _VREM)])
    slab_wait(0)
    extract(0)
    plsc.subcore_barrier()
    slab_start(1)

    def gather_start(rv, sem, j):
        pltpu.async_copy(xs.at[idx1.at[pl.ds(j * _C, _C)]], rv, sem)

    def gather_wait(rv, sem, j):
        pltpu.make_async_copy(xs.at[idx1.at[pl.ds(j * _C, _C)]], rv,
                              sem).wait()

    def store_start(rv, sem, j):
        pltpu.async_copy(rv, out_hbm.at[pl.ds(base + j * _C, _C)], sem)

    def store_wait(rv, sem, j):
        pltpu.make_async_copy(rv, out_hbm.at[pl.ds(base + j * _C, _C)],
                              sem).wait()

    gather_start(r0, g0, 0)
    for p in range(_PH):

        def pair(g, carry, _p=p):
            j0 = _p * _CPP + 2 * g
            gather_wait(r0, g0, j0)
            store_start(r0, s0, j0)
            @pl.when(g > 0)
            def _():
                store_wait(r1, s1, j0 - 1)
            gather_start(r1, g1, j0 + 1)
            gather_wait(r1, g1, j0 + 1)
            store_start(r1, s1, j0 + 1)
            store_wait(r0, s0, j0)
            @pl.when(g < _PPP - 1)
            def _():
                gather_start(r0, g0, j0 + 2)
            return carry

        lax.fori_loop(0, _PPP, pair, 0)
        if p + 1 < _PH:
            slab_wait(p + 1)
            extract(p + 1)
            if p + 2 < _PH:
                slab_start(p + 2)
            gather_start(r0, g0, (p + 1) * _CPP)
            store_wait(r1, s1, (p + 1) * _CPP - 1)
    store_wait(r1, s1, _PH * _CPP - 1)

    @pl.when(wid < 4)
    def _():
        src = ei_hbm.at[pl.ds(0, 2), pl.ds(_XBASE, _XTOT)]
        dst = ib.at[pl.ds(0, 2), pl.ds(0, _XTOT)]
        pltpu.async_copy(src, dst, isem)
        pltpu.make_async_copy(src, dst, isem).wait()
        for k in range(_XTOT // _L):
            idx1[pl.ds(k * _L, _L)] = ib[0, pl.ds(k * _L, _L)]
        pltpu.async_copy(
            xs.at[idx1.at[pl.ds(wid * _C, _C)]], r0, g0).wait()
        pltpu.sync_copy(r0, out_hbm.at[pl.ds(_XBASE + wid * _C, _C)])


@jax.jit
def _gather(ei, x):
    mesh = plsc.VectorSubcoreMesh(core_axis_name="c", subcore_axis_name="s")
    run = pl.kernel(
        _gather_body,
        mesh=mesh,
        out_type=jax.ShapeDtypeStruct((_B, _D), jnp.float32),
        scratch_types=[
            pltpu.VMEM_SHARED((_V, _D), jnp.float32),
            pltpu.VMEM((2, _PHE), jnp.int32),
            pltpu.VMEM((_KW,), jnp.int32),
            pltpu.VMEM((_C, _D), jnp.float32),
            pltpu.VMEM((_C, _D), jnp.float32),
            pltpu.SemaphoreType.DMA,
            pltpu.SemaphoreType.DMA,
            pltpu.SemaphoreType.DMA,
            pltpu.SemaphoreType.DMA,
            pltpu.SemaphoreType.DMA,
        ],
    )
    return run(ei, x)


def kernel(x, edge_index):
    ei = edge_index if edge_index.dtype == jnp.int32 else (
        edge_index.astype(jnp.int32))
    messages = _gather(ei, x)
    return (x, edge_index, messages)


# speedup vs baseline: 1.0099x; 1.0099x over previous
import jax
import jax.numpy as jnp
from jax import lax
from jax.experimental import pallas as pl
from jax.experimental.pallas import tpu as pltpu
from jax.experimental.pallas import tpu_sc as plsc

_B = 320000
_D = 128
_V = 10000
_NC = 2
_NS = 16
_NW = _NC * _NS

_KW = 9984
_C = 128
_PH = 3
_PHE = _KW // _PH
_CPP = _PHE // _C
_PPP = _CPP // 2

_XBASE = _NW * _KW
_XTOT = _B - _XBASE

_VPT = 624
_VREM = _V - _NS * _VPT

_L = 16


def _gather_body(ei_hbm, x_hbm, out_hbm, xs, ib, idx1, r0, r1,
                 g0, g1, s0, s1, isem):
    cid = lax.axis_index("c")
    sid = lax.axis_index("s")
    wid = sid * _NC + cid
    base = wid * _KW

    def slab_start(p):
        src = ei_hbm.at[pl.ds(0, 2), pl.ds(base + p * _PHE, _PHE)]
        pltpu.async_copy(src, ib, isem)

    def slab_wait(p):
        src = ei_hbm.at[pl.ds(0, 2), pl.ds(base + p * _PHE, _PHE)]
        pltpu.make_async_copy(src, ib, isem).wait()

    def extract(p):
        zb = p * _PHE

        def cp(k, carry):
            for u in range(8):
                off = k * (8 * _L) + u * _L
                idx1[pl.ds(zb + off, _L)] = ib[0, pl.ds(off, _L)]
            return carry

        lax.fori_loop(0, _PHE // (8 * _L), cp, 0)

    slab_start(0)

    pltpu.sync_copy(x_hbm.at[pl.ds(sid * _VPT, _VPT)],
                    xs.at[pl.ds(sid * _VPT, _VPT)])
    @pl.when(sid == 0)
    def _():
        pltpu.sync_copy(x_hbm.at[pl.ds(_NS * _VPT, _VREM)],
                        xs.at[pl.ds(_NS * _VPT, _VREM)])
    slab_wait(0)
    extract(0)
    plsc.subcore_barrier()
    slab_start(1)

    def gather_start(rv, sem, j):
        pltpu.async_copy(xs.at[idx1.at[pl.ds(j * _C, _C)]], rv, sem)

    def gather_wait(rv, sem, j):
        pltpu.make_async_copy(xs.at[idx1.at[pl.ds(j * _C, _C)]], rv,
                              sem).wait()

    def store_start(rv, sem, j):
        pltpu.async_copy(rv, out_hbm.at[pl.ds(base + j * _C, _C)], sem)

    def store_wait(rv, sem, j):
        pltpu.make_async_copy(rv, out_hbm.at[pl.ds(base + j * _C, _C)],
                              sem).wait()

    gather_start(r0, g0, 0)
    for p in range(_PH):

        def pair(g, carry, _p=p):
            j0 = _p * _CPP + 2 * g
            gather_wait(r0, g0, j0)
            store_start(r0, s0, j0)
            @pl.when(g > 0)
            def _():
                store_wait(r1, s1, j0 - 1)
            gather_start(r1, g1, j0 + 1)
            gather_wait(r1, g1, j0 + 1)
            store_start(r1, s1, j0 + 1)
            store_wait(r0, s0, j0)
            @pl.when(g < _PPP - 1)
            def _():
                gather_start(r0, g0, j0 + 2)
            return carry

        lax.fori_loop(0, _PPP, pair, 0)
        if p + 1 < _PH:
            slab_wait(p + 1)
            extract(p + 1)
            if p + 2 < _PH:
                slab_start(p + 2)
            gather_start(r0, g0, (p + 1) * _CPP)
            store_wait(r1, s1, (p + 1) * _CPP - 1)
    store_wait(r1, s1, _PH * _CPP - 1)

    @pl.when(wid < 4)
    def _():
        src = ei_hbm.at[pl.ds(0, 2), pl.ds(_XBASE, _XTOT)]
        dst = ib.at[pl.ds(0, 2), pl.ds(0, _XTOT)]
        pltpu.async_copy(src, dst, isem)
        pltpu.make_async_copy(src, dst, isem).wait()
        for k in range(_XTOT // _L):
            idx1[pl.ds(k * _L, _L)] = ib[0, pl.ds(k * _L, _L)]
        pltpu.async_copy(
            xs.at[idx1.at[pl.ds(wid * _C, _C)]], r0, g0).wait()
        pltpu.sync_copy(r0, out_hbm.at[pl.ds(_XBASE + wid * _C, _C)])


@jax.jit
def _gather(ei, x):
    mesh = plsc.VectorSubcoreMesh(core_axis_name="c", subcore_axis_name="s")
    run = pl.kernel(
        _gather_body,
        mesh=mesh,
        out_type=jax.ShapeDtypeStruct((_B, _D), jnp.float32),
        scratch_types=[
            pltpu.VMEM_SHARED((_V, _D), jnp.float32),
            pltpu.VMEM((2, _PHE), jnp.int32),
            pltpu.VMEM((_KW,), jnp.int32),
            pltpu.VMEM((_C, _D), jnp.float32),
            pltpu.VMEM((_C, _D), jnp.float32),
            pltpu.SemaphoreType.DMA,
            pltpu.SemaphoreType.DMA,
            pltpu.SemaphoreType.DMA,
            pltpu.SemaphoreType.DMA,
            pltpu.SemaphoreType.DMA,
        ],
    )
    return run(ei, x)


def kernel(x, edge_index):
    ei = edge_index if edge_index.dtype == jnp.int32 else (
        edge_index.astype(jnp.int32))
    messages = _gather(ei, x)
    return (x, edge_index, messages)
